# Initial kernel scaffold; baseline (speedup 1.0000x reference)
import functools

import jax
import jax.numpy as jnp
from jax import lax
from jax.experimental import pallas as pl
from jax.experimental.pallas import tpu as pltpu
from jax.experimental.pallas import tpu_sc as plsc

N = 10000
D_IN = 128
H = 4
C = 128
HC = H * C
BG = 64
EMBED = 10

N_PAD = 10112
DL = 144
DR = 128
NTILES = 16
EB = 128
ROWS_PER_TILE = N_PAD // NTILES
NBLK_PREP = N_PAD // 128


def _prep_body(x_ref, wl_ref, wr_ref, tl_ref, tr_ref):
    xb = x_ref[...]
    yl = jnp.dot(xb, wl_ref[...], preferred_element_type=jnp.float32)
    yr = jnp.dot(xb, wr_ref[...], preferred_element_type=jnp.float32)
    ones_col = jnp.where(
        lax.broadcasted_iota(jnp.int32, (128, 16), 1) == 0, 1.0, 0.0
    ).astype(jnp.float32)
    for h in range(H):
        tl_ref[h, :, 0:128] = yl[:, h * 128:(h + 1) * 128]
        tl_ref[h, :, 128:144] = ones_col
        tr_ref[h, :, :] = yr[:, h * 128:(h + 1) * 128]


def _prep(x_pad, W_l, W_r):
    return pl.pallas_call(
        _prep_body,
        grid=(NBLK_PREP,),
        in_specs=[
            pl.BlockSpec((128, D_IN), lambda i: (i, 0)),
            pl.BlockSpec((D_IN, HC), lambda i: (0, 0)),
            pl.BlockSpec((D_IN, HC), lambda i: (0, 0)),
        ],
        out_specs=[
            pl.BlockSpec((H, 128, DL), lambda i: (0, i, 0)),
            pl.BlockSpec((H, 128, DR), lambda i: (0, i, 0)),
        ],
        out_shape=[
            jax.ShapeDtypeStruct((H, N_PAD, DL), jnp.float32),
            jax.ShapeDtypeStruct((H, N_PAD, DR), jnp.float32),
        ],
    )(x_pad, W_l, W_r)


def _sc_edge_kernel(ept, nb):
    mesh = plsc.VectorSubcoreMesh(
        core_axis_name="c", subcore_axis_name="s", num_cores=2,
        num_subcores=NTILES)

    @functools.partial(
        pl.kernel,
        mesh=mesh,
        out_type=jax.ShapeDtypeStruct((H * N_PAD, DL), jnp.float32),
        scratch_types=[
            pltpu.VMEM_SHARED((N_PAD, DL), jnp.float32),
            pltpu.VMEM((EB, DL), jnp.float32),
            pltpu.VMEM((EB, DL), jnp.float32),
            pltpu.VMEM((EB, DR), jnp.float32),
            pltpu.VMEM((EB, DR), jnp.float32),
            pltpu.VMEM((None,), jnp.int32),
            pltpu.VMEM((None,), jnp.int32),
            pltpu.VMEM((EB,), jnp.int32),
            pltpu.VMEM((EB,), jnp.int32),
            pltpu.VMEM((EB,), jnp.int32),
            pltpu.VMEM((EB,), jnp.int32),
            pltpu.VMEM((EB,), jnp.int32),
            pltpu.VMEM((EB,), jnp.int32),
            pltpu.VMEM((8, 16), jnp.float32),
            pltpu.SemaphoreType.DMA,
            pltpu.SemaphoreType.DMA,
            pltpu.SemaphoreType.DMA,
            pltpu.SemaphoreType.DMA,
        ],
    )
    def sc_kernel(tl_hbm, tr_hbm, src_hbm, dst_hbm, att_hbm, out_hbm,
                  accum, l0, l1, r0, r1, src_all, dst_all,
                  so0, so1, do0, do1, db0, db1, attb,
                  sl0, sl1, sr0, sr1):
        cid = lax.axis_index("c")
        sid = lax.axis_index("s")
        lbuf = (l0, l1)
        rbuf = (r0, r1)
        sobuf = (so0, so1)
        dobuf = (do0, do1)
        dbbuf = (db0, db1)
        sem_l = (sl0, sl1)
        sem_r = (sr0, sr1)

        ebase = sid * ept
        pltpu.sync_copy(src_hbm.at[pl.ds(ebase, ept)], src_all)
        pltpu.sync_copy(dst_hbm.at[pl.ds(ebase, ept)], dst_all)
        rbase = sid * ROWS_PER_TILE

        def fill_idx(dst_ref, idx_all, blk, off):
            for k in range(EB // 16):
                v = idx_all[pl.ds(blk * EB + k * 16, 16)]
                dst_ref[pl.ds(k * 16, 16)] = v + off

        def issue_gathers(blk, b, hoff):
            fill_idx(sobuf[b], src_all, blk, hoff)
            fill_idx(dobuf[b], dst_all, blk, hoff)
            fill_idx(dbbuf[b], dst_all, blk, 0)
            pltpu.async_copy(tl_hbm.at[sobuf[b]], lbuf[b], sem_l[b])
            pltpu.async_copy(tr_hbm.at[dobuf[b]], rbuf[b], sem_r[b])

        def compute_block(b):
            avecs = [attb[k, :] for k in range(8)]

            def ebody(be, carry):
                lv = [lbuf[b][be, pl.ds(16 * k, 16)] for k in range(9)]
                acc = None
                for k in range(8):
                    t = lv[k] + rbuf[b][be, pl.ds(16 * k, 16)]
                    y = jnp.maximum(t, 0.2 * t)
                    p = y * avecs[k]
                    acc = p if acc is None else acc + p
                alpha = jnp.sum(acc)
                exv = jnp.exp(jnp.broadcast_to(alpha, (16,)))
                for k in range(9):
                    lbuf[b][be, pl.ds(16 * k, 16)] = lv[k] * exv
                return carry

            lax.fori_loop(0, EB, ebody, 0, unroll=2)

        zvec = jnp.zeros((16,), jnp.float32)

        for rnd in range(2):
            h = 2 * rnd + cid
            hoff = h * N_PAD

            def zbody(i, carry):
                for k in range(9):
                    l0[i, pl.ds(16 * k, 16)] = zvec
                return carry
            lax.fori_loop(0, EB, zbody, 0)
            for k in range(ROWS_PER_TILE // EB):
                pltpu.sync_copy(l0.at[pl.ds(0, EB)],
                                accum.at[pl.ds(rbase + k * EB, EB)])
            rem = ROWS_PER_TILE % EB
            if rem:
                pltpu.sync_copy(
                    l0.at[pl.ds(0, rem)],
                    accum.at[pl.ds(rbase + (ROWS_PER_TILE // EB) * EB, rem)])
            pltpu.sync_copy(att_hbm.at[pl.ds(h * 8, 8)], attb)
            plsc.subcore_barrier()

            issue_gathers(0, 0, hoff)
            issue_gathers(1, 1, hoff)

            def gbody(g, carry):
                for b in range(2):
                    blk = 2 * g + b
                    pltpu.make_async_copy(
                        tl_hbm.at[sobuf[b]], lbuf[b], sem_l[b]).wait()
                    pltpu.make_async_copy(
                        tr_hbm.at[dobuf[b]], rbuf[b], sem_r[b]).wait()
                    compute_block(b)
                    pltpu.sync_copy(lbuf[b], accum.at[dbbuf[b]], add=True)

                    @pl.when(g < nb // 2 - 1)
                    def _():
                        issue_gathers(blk + 2, b, hoff)
                return carry

            lax.fori_loop(0, nb // 2, gbody, 0)
            plsc.subcore_barrier()
            pltpu.sync_copy(
                accum.at[pl.ds(rbase, ROWS_PER_TILE)],
                out_hbm.at[pl.ds(hoff + rbase, ROWS_PER_TILE)])

    return sc_kernel


POST_R = 1264
POST_G = N_PAD // POST_R


def _post_body(acc_ref, batch_ref, bias_ref, w1_ref, b1_ref, w2_ref, b2_ref,
               out_ref, pooled_s, cnt_s):
    i = pl.program_id(0)

    @pl.when(i == 0)
    def _():
        pooled_s[...] = jnp.zeros_like(pooled_s)
        cnt_s[...] = jnp.zeros_like(cnt_s)

    b = batch_ref[...]
    gid = lax.broadcasted_iota(jnp.int32, (1, BG), 1).astype(jnp.float32)
    P = (b == gid).astype(jnp.float32)
    parts = []
    for h in range(H):
        den = acc_ref[h, :, 128:129]
        parts.append(acc_ref[h, :, 0:128] / jnp.maximum(den, 1e-30))
    nodes = jnp.concatenate(parts, axis=1)
    pooled_s[...] += lax.dot_general(
        P, nodes, (((0,), (0,)), ((), ())), preferred_element_type=jnp.float32)
    cnt_s[...] += jnp.sum(P, axis=0, keepdims=True)

    @pl.when(i == POST_G - 1)
    def _():
        cnt = jnp.maximum(cnt_s[...], 1.0)
        pm = pooled_s[...] / cnt.reshape(BG, 1) + bias_ref[...]
        hmid = jnp.maximum(
            jnp.dot(pm, w1_ref[...], preferred_element_type=jnp.float32)
            + b1_ref[...], 0.0)
        out_ref[...] = (
            jnp.dot(hmid, w2_ref[...], preferred_element_type=jnp.float32)
            + b2_ref[...])


def _post(acc, batch_f, bias, mlp_W1, mlp_b1, mlp_W2, mlp_b2):
    return pl.pallas_call(
        _post_body,
        grid=(POST_G,),
        in_specs=[
            pl.BlockSpec((H, POST_R, DL), lambda i: (0, i, 0)),
            pl.BlockSpec((POST_R, 1), lambda i: (i, 0)),
            pl.BlockSpec((1, HC), lambda i: (0, 0)),
            pl.BlockSpec((HC, C), lambda i: (0, 0)),
            pl.BlockSpec((1, C), lambda i: (0, 0)),
            pl.BlockSpec((C, EMBED), lambda i: (0, 0)),
            pl.BlockSpec((1, EMBED), lambda i: (0, 0)),
        ],
        out_specs=pl.BlockSpec((BG, EMBED), lambda i: (0, 0)),
        out_shape=jax.ShapeDtypeStruct((BG, EMBED), jnp.float32),
        scratch_shapes=[
            pltpu.VMEM((BG, HC), jnp.float32),
            pltpu.VMEM((1, BG), jnp.float32),
        ],
    )(acc, batch_f, bias, mlp_W1, mlp_b1, mlp_W2, mlp_b2)


def kernel(x, edge_index, batch, W_l, W_r, att, bias,
           mlp_W1, mlp_b1, mlp_W2, mlp_b2):
    E = edge_index.shape[1]
    Et = E + N
    ept = -(-Et // (NTILES * 2 * EB)) * 2 * EB
    Et_pad = ept * NTILES
    nb = ept // EB

    loop = jnp.arange(N, dtype=jnp.int32)
    pad = jnp.full((Et_pad - Et,), N, dtype=jnp.int32)
    src = jnp.concatenate([edge_index[0].astype(jnp.int32), loop, pad])
    dst = jnp.concatenate([edge_index[1].astype(jnp.int32), loop, pad])

    x_pad = jnp.zeros((N_PAD, D_IN), jnp.float32).at[:N].set(x)
    TL, TR = _prep(x_pad, W_l, W_r)
    TL2 = TL.reshape(H * N_PAD, DL)
    TR2 = TR.reshape(H * N_PAD, DR)
    att_r = att.reshape(H * 8, 16)

    acc = _sc_edge_kernel(ept, nb)(TL2, TR2, src, dst, att_r)
    acc = acc.reshape(H, N_PAD, DL)

    batch_f = jnp.full((N_PAD, 1), -1.0, jnp.float32).at[:N, 0].set(
        batch.astype(jnp.float32))
    return _post(acc, batch_f, bias.reshape(1, HC), mlp_W1,
                 mlp_b1.reshape(1, C), mlp_W2, mlp_b2.reshape(1, EMBED))


# speedup vs baseline: 18.1942x; 18.1942x over previous
import functools

import jax
import jax.numpy as jnp
from jax import lax
from jax.experimental import pallas as pl
from jax.experimental.pallas import tpu as pltpu
from jax.experimental.pallas import tpu_sc as plsc

N = 10000
D_IN = 128
H = 4
C = 128
HC = H * C
BG = 64
EMBED = 10

N_PAD = 10112
DL = 144
DR = 128
NTILES = 16
EB = 64
ROWS_PER_TILE = N_PAD // NTILES
NBLK_PREP = N_PAD // 128


def _prep_body(x_ref, wl_ref, wr_ref, tl_ref, tr_ref):
    xb = x_ref[...]
    yl = jnp.dot(xb, wl_ref[...], preferred_element_type=jnp.float32)
    yr = jnp.dot(xb, wr_ref[...], preferred_element_type=jnp.float32)
    ones_col = jnp.where(
        lax.broadcasted_iota(jnp.int32, (128, 16), 1) == 0, 1.0, 0.0
    ).astype(jnp.float32)
    for h in range(H):
        tl_ref[h, :, 0:128] = yl[:, h * 128:(h + 1) * 128]
        tl_ref[h, :, 128:144] = ones_col
        tr_ref[h, :, :] = yr[:, h * 128:(h + 1) * 128]


def _prep(x_pad, W_l, W_r):
    return pl.pallas_call(
        _prep_body,
        grid=(NBLK_PREP,),
        in_specs=[
            pl.BlockSpec((128, D_IN), lambda i: (i, 0)),
            pl.BlockSpec((D_IN, HC), lambda i: (0, 0)),
            pl.BlockSpec((D_IN, HC), lambda i: (0, 0)),
        ],
        out_specs=[
            pl.BlockSpec((H, 128, DL), lambda i: (0, i, 0)),
            pl.BlockSpec((H, 128, DR), lambda i: (0, i, 0)),
        ],
        out_shape=[
            jax.ShapeDtypeStruct((H, N_PAD, DL), jnp.float32),
            jax.ShapeDtypeStruct((H, N_PAD, DR), jnp.float32),
        ],
    )(x_pad, W_l, W_r)


def _sc_edge_kernel(ept, nb):
    mesh = plsc.VectorSubcoreMesh(
        core_axis_name="c", subcore_axis_name="s", num_cores=2,
        num_subcores=NTILES)

    @functools.partial(
        pl.kernel,
        mesh=mesh,
        compiler_params=pltpu.CompilerParams(
            needs_layout_passes=False, use_tc_tiling_on_sc=False),
        out_type=jax.ShapeDtypeStruct((H * N_PAD, DL), jnp.float32),
        scratch_types=[
            pltpu.VMEM_SHARED((N_PAD, DL), jnp.float32),
            pltpu.VMEM((EB, DL), jnp.float32),
            pltpu.VMEM((EB, DL), jnp.float32),
            pltpu.VMEM((EB, DR), jnp.float32),
            pltpu.VMEM((EB, DR), jnp.float32),
            pltpu.VMEM((EB,), jnp.int32),
            pltpu.VMEM((EB,), jnp.int32),
            pltpu.VMEM((EB,), jnp.int32),
            pltpu.VMEM((EB,), jnp.int32),
            pltpu.VMEM((EB,), jnp.int32),
            pltpu.VMEM((EB,), jnp.int32),
            pltpu.VMEM((8, 16), jnp.float32),
            pltpu.SemaphoreType.DMA,
            pltpu.SemaphoreType.DMA,
            pltpu.SemaphoreType.DMA,
            pltpu.SemaphoreType.DMA,
            pltpu.SemaphoreType.DMA,
            pltpu.SemaphoreType.DMA,
        ],
    )
    def sc_kernel(tl_hbm, tr_hbm, srcoff_hbm, dstoff_hbm, draw_hbm, att_hbm,
                  out_hbm,
                  accum, l0, l1, r0, r1,
                  so0, so1, do0, do1, db0, db1, attb,
                  sl0, sl1, sr0, sr1, si0, si1):
        cid = lax.axis_index("c")
        sid = lax.axis_index("s")
        lbuf = (l0, l1)
        rbuf = (r0, r1)
        sobuf = (so0, so1)
        dobuf = (do0, do1)
        dbbuf = (db0, db1)
        sem_l = (sl0, sl1)
        sem_r = (sr0, sr1)
        sem_i = (si0, si1)
        rbase = sid * ROWS_PER_TILE

        def issue_idx_gather(rc, blk, b):
            base = ((rc * NTILES + sid) * nb + blk) * EB
            pltpu.async_copy(srcoff_hbm.at[pl.ds(base, EB)], sobuf[b],
                             sem_i[b])
            pltpu.async_copy(dstoff_hbm.at[pl.ds(base, EB)], dobuf[b],
                             sem_i[b])

        def issue_idx_draw(blk, b):
            dbase = (sid * nb + blk) * EB
            pltpu.async_copy(draw_hbm.at[pl.ds(dbase, EB)], dbbuf[b],
                             sem_i[b])

        def issue_idx(rc, blk, b):
            issue_idx_gather(rc, blk, b)
            issue_idx_draw(blk, b)

        def wait_idx(b):
            pltpu.make_async_copy(srcoff_hbm.at[pl.ds(0, EB)], sobuf[b],
                                  sem_i[b]).wait()
            pltpu.make_async_copy(dstoff_hbm.at[pl.ds(0, EB)], dobuf[b],
                                  sem_i[b]).wait()
            pltpu.make_async_copy(draw_hbm.at[pl.ds(0, EB)], dbbuf[b],
                                  sem_i[b]).wait()

        def issue_gathers(b):
            pltpu.async_copy(tl_hbm.at[sobuf[b]], lbuf[b], sem_l[b])
            pltpu.async_copy(tr_hbm.at[dobuf[b]], rbuf[b], sem_r[b])

        def wait_gathers(b):
            pltpu.make_async_copy(tl_hbm.at[sobuf[b]], lbuf[b],
                                  sem_l[b]).wait()
            pltpu.make_async_copy(tr_hbm.at[dobuf[b]], rbuf[b],
                                  sem_r[b]).wait()

        def compute_block(b):
            avecs = [attb[k, :] for k in range(8)]

            def ebody(be, carry):
                lv = [lbuf[b][be, pl.ds(16 * k, 16)] for k in range(9)]
                acc = None
                for k in range(8):
                    t = lv[k] + rbuf[b][be, pl.ds(16 * k, 16)]
                    y = jnp.maximum(t, 0.2 * t)
                    p = y * avecs[k]
                    acc = p if acc is None else acc + p
                alpha = jnp.sum(acc)
                exv = jnp.exp(jnp.broadcast_to(alpha, (16,)))
                for k in range(9):
                    lbuf[b][be, pl.ds(16 * k, 16)] = lv[k] * exv
                return carry

            lax.fori_loop(0, EB, ebody, 0, unroll=2)

        zvec = jnp.zeros((16,), jnp.float32)

        for rnd in range(2):
            h = 2 * rnd + cid
            rc = 2 * rnd + cid

            def zbody(i, carry):
                for k in range(9):
                    l0[i, pl.ds(16 * k, 16)] = zvec
                return carry
            lax.fori_loop(0, EB, zbody, 0)
            for k in range(ROWS_PER_TILE // EB):
                pltpu.sync_copy(l0.at[pl.ds(0, EB)],
                                accum.at[pl.ds(rbase + k * EB, EB)])
            rem = ROWS_PER_TILE % EB
            if rem:
                pltpu.sync_copy(
                    l0.at[pl.ds(0, rem)],
                    accum.at[pl.ds(rbase + (ROWS_PER_TILE // EB) * EB, rem)])
            pltpu.sync_copy(att_hbm.at[pl.ds(h * 8, 8)], attb)
            plsc.subcore_barrier()

            issue_idx(rc, 0, 0)
            issue_idx(rc, 1, 1)
            wait_idx(0)
            issue_gathers(0)

            def gbody(g, carry):
                not_last = g < nb // 2 - 1
                for b in range(2):
                    wait_gathers(b)

                    def next_stage(b=b):
                        wait_idx(1 - b)
                        issue_gathers(1 - b)

                    if b == 0:
                        next_stage()
                    else:
                        pl.when(not_last)(next_stage)
                    pl.when(not_last)(
                        lambda b=b, g=g: issue_idx_gather(rc, 2 * g + b + 2, b))
                    compute_block(b)
                    pltpu.sync_copy(lbuf[b], accum.at[dbbuf[b]], add=True)
                    pl.when(not_last)(
                        lambda b=b, g=g: issue_idx_draw(2 * g + b + 2, b))
                return carry

            lax.fori_loop(0, nb // 2, gbody, 0)
            plsc.subcore_barrier()
            pltpu.sync_copy(
                accum.at[pl.ds(rbase, ROWS_PER_TILE)],
                out_hbm.at[pl.ds(h * N_PAD + rbase, ROWS_PER_TILE)])

    return sc_kernel


POST_R = 1264
POST_G = N_PAD // POST_R


def _post_body(acc_ref, batch_ref, bias_ref, w1_ref, b1_ref, w2_ref, b2_ref,
               out_ref, pooled_s, cnt_s):
    i = pl.program_id(0)

    @pl.when(i == 0)
    def _():
        pooled_s[...] = jnp.zeros_like(pooled_s)
        cnt_s[...] = jnp.zeros_like(cnt_s)

    b = batch_ref[...]
    gid = lax.broadcasted_iota(jnp.int32, (1, BG), 1).astype(jnp.float32)
    P = (b == gid).astype(jnp.float32)
    parts = []
    for h in range(H):
        den = acc_ref[h, :, 128:129]
        parts.append(acc_ref[h, :, 0:128] / jnp.maximum(den, 1e-30))
    nodes = jnp.concatenate(parts, axis=1)
    pooled_s[...] += lax.dot_general(
        P, nodes, (((0,), (0,)), ((), ())), preferred_element_type=jnp.float32)
    cnt_s[...] += jnp.sum(P, axis=0, keepdims=True)

    @pl.when(i == POST_G - 1)
    def _():
        cnt = jnp.maximum(cnt_s[...], 1.0)
        pm = pooled_s[...] / cnt.reshape(BG, 1) + bias_ref[...]
        hmid = jnp.maximum(
            jnp.dot(pm, w1_ref[...], preferred_element_type=jnp.float32)
            + b1_ref[...], 0.0)
        out_ref[...] = (
            jnp.dot(hmid, w2_ref[...], preferred_element_type=jnp.float32)
            + b2_ref[...])


def _post(acc, batch_f, bias, mlp_W1, mlp_b1, mlp_W2, mlp_b2):
    return pl.pallas_call(
        _post_body,
        grid=(POST_G,),
        in_specs=[
            pl.BlockSpec((H, POST_R, DL), lambda i: (0, i, 0)),
            pl.BlockSpec((POST_R, 1), lambda i: (i, 0)),
            pl.BlockSpec((1, HC), lambda i: (0, 0)),
            pl.BlockSpec((HC, C), lambda i: (0, 0)),
            pl.BlockSpec((1, C), lambda i: (0, 0)),
            pl.BlockSpec((C, EMBED), lambda i: (0, 0)),
            pl.BlockSpec((1, EMBED), lambda i: (0, 0)),
        ],
        out_specs=pl.BlockSpec((BG, EMBED), lambda i: (0, 0)),
        out_shape=jax.ShapeDtypeStruct((BG, EMBED), jnp.float32),
        scratch_shapes=[
            pltpu.VMEM((BG, HC), jnp.float32),
            pltpu.VMEM((1, BG), jnp.float32),
        ],
    )(acc, batch_f, bias, mlp_W1, mlp_b1, mlp_W2, mlp_b2)


def kernel(x, edge_index, batch, W_l, W_r, att, bias,
           mlp_W1, mlp_b1, mlp_W2, mlp_b2):
    E = edge_index.shape[1]
    Et = E + N
    ept = -(-Et // (NTILES * 2 * EB)) * 2 * EB
    Et_pad = ept * NTILES
    nb = ept // EB

    loop = jnp.arange(N, dtype=jnp.int32)
    pad = jnp.full((Et_pad - Et,), N, dtype=jnp.int32)
    src = jnp.concatenate([edge_index[0].astype(jnp.int32), loop, pad])
    dst = jnp.concatenate([edge_index[1].astype(jnp.int32), loop, pad])

    hoffs = (jnp.arange(4, dtype=jnp.int32) * N_PAD)[:, None]
    srcoff = (hoffs + src[None, :]).reshape(-1)
    dstoff = (hoffs + dst[None, :]).reshape(-1)

    x_pad = jnp.zeros((N_PAD, D_IN), jnp.float32).at[:N].set(x)
    TL, TR = _prep(x_pad, W_l, W_r)
    TL2 = TL.reshape(H * N_PAD, DL)
    TR2 = TR.reshape(H * N_PAD, DR)
    att_r = att.reshape(H * 8, 16)

    acc = _sc_edge_kernel(ept, nb)(TL2, TR2, srcoff, dstoff, dst, att_r)
    acc = acc.reshape(H, N_PAD, DL)

    batch_f = jnp.full((N_PAD, 1), -1.0, jnp.float32).at[:N, 0].set(
        batch.astype(jnp.float32))
    return _post(acc, batch_f, bias.reshape(1, HC), mlp_W1,
                 mlp_b1.reshape(1, C), mlp_W2, mlp_b2.reshape(1, EMBED))
